# Initial kernel scaffold; baseline (speedup 1.0000x reference)
import jax
import jax.numpy as jnp
from jax.experimental import pallas as pl


def kernel(node_features, edge_features, edge_index, batch, W_embed, b_embed, W_conv, b_conv, W_lin, b_lin):
    raise NotImplementedError("write your pallas kernel here")


# speedup vs baseline: 14.5999x; 14.5999x over previous
import functools

import jax
import jax.numpy as jnp
from jax import lax
from jax.experimental import pallas as pl
from jax.experimental.pallas import tpu as pltpu
from jax.experimental.pallas import tpu_sc as plsc

_N = 10000
_D = 128
_G = 64
_NC = 2
_NS = 16
_NW = _NC * _NS
_CH = 128
_CHB = 80
_EPW = _CH * _CHB
_EPAD = _NW * _EPW
_NPAD = 10240
_RPT = _NPAD // _NS
_NR = _NPAD // _CH

_mesh = plsc.VectorSubcoreMesh(core_axis_name="c", subcore_axis_name="s")


@functools.partial(
    pl.kernel,
    out_type=jax.ShapeDtypeStruct((_NC, _NS, _NPAD), jnp.float32),
    mesh=_mesh,
    scratch_types=[
        pltpu.VMEM((_CHB, _CH), jnp.int32),
        pltpu.VMEM((_NPAD,), jnp.float32),
    ],
    compiler_params=pltpu.CompilerParams(needs_layout_passes=False),
)
def _deg_kernel(dst_hbm, zeros_hbm, out_hbm, dst_v, deg_v):
    c = lax.axis_index("c")
    s = lax.axis_index("s")
    pltpu.sync_copy(zeros_hbm, deg_v)
    pltpu.sync_copy(dst_hbm.at[c, s], dst_v)
    ones = jnp.ones((16,), jnp.float32)

    def body(j, carry):
        def inner(k, carry2):
            idx = dst_v[j, pl.ds(k * 16, 16)]
            plsc.addupdate_scatter(deg_v, [idx], ones)
            return carry2

        return lax.fori_loop(0, _CH // 16, inner, carry)

    lax.fori_loop(0, _CHB, body, 0)
    pltpu.sync_copy(deg_v, out_hbm.at[c, s])


@functools.partial(
    pl.kernel,
    out_type=jax.ShapeDtypeStruct((_NC, _NPAD, _D), jnp.float32),
    mesh=_mesh,
    scratch_types=[
        pltpu.VMEM((_CHB, _CH), jnp.int32),
        pltpu.VMEM((_CHB, _CH), jnp.int32),
        pltpu.VMEM((_CH, _D), jnp.float32),
        pltpu.VMEM_SHARED((_NPAD, _D), jnp.float32),
        pltpu.SemaphoreType.DMA,
    ],
)
def _agg_kernel(y_hbm, src_hbm, dst_hbm, zeros_hbm, out_hbm,
                src_v, dst_v, rows_v, agg_sh, sem):
    c = lax.axis_index("c")
    s = lax.axis_index("s")
    pltpu.sync_copy(zeros_hbm.at[pl.ds(s * _RPT, _RPT)],
                    agg_sh.at[pl.ds(s * _RPT, _RPT)])
    pltpu.sync_copy(src_hbm.at[c, s], src_v)
    pltpu.sync_copy(dst_hbm.at[c, s], dst_v)
    plsc.subcore_barrier()

    def body(j, carry):
        pltpu.async_copy(y_hbm.at[src_v.at[j]], rows_v, sem).wait()
        pltpu.sync_copy(rows_v, agg_sh.at[dst_v.at[j]], add=True)
        return carry

    lax.fori_loop(0, _CHB, body, 0)
    plsc.subcore_barrier()
    pltpu.sync_copy(agg_sh.at[pl.ds(s * _RPT, _RPT)],
                    out_hbm.at[c, pl.ds(s * _RPT, _RPT)])


def _dense_body(nf_ref, we_ref, be_ref, wc_ref, dp_ref, xw_ref, dinv_ref):
    x = jnp.maximum(
        jnp.dot(nf_ref[...], we_ref[...], preferred_element_type=jnp.float32)
        + be_ref[...], 0.0)
    xw_ref[...] = jnp.dot(x, wc_ref[...], preferred_element_type=jnp.float32)
    degf = jnp.sum(dp_ref[...], axis=0)
    dinv_ref[...] = lax.rsqrt(degf + 1.0)


_dense = pl.pallas_call(
    _dense_body,
    out_shape=(
        jax.ShapeDtypeStruct((_N, _D), jnp.float32),
        jax.ShapeDtypeStruct((_NR, _CH), jnp.float32),
    ),
)


def _scale_body(xw_ref, dinv_ref, y_ref):
    y_ref[...] = xw_ref[...] * dinv_ref[...]


_scale = pl.pallas_call(
    _scale_body,
    out_shape=jax.ShapeDtypeStruct((_N, _D), jnp.float32),
)


def _post_body(a0_ref, a1_ref, xw_ref, dinv_ref, bc_ref, batch_ref,
               wl_ref, bl_ref, out_ref):
    agg = a0_ref[:_N, :] + a1_ref[:_N, :]
    dinv = dinv_ref[...]
    x2 = jnp.maximum(dinv * agg + dinv * dinv * xw_ref[...] + bc_ref[...], 0.0)
    bi = lax.broadcasted_iota(jnp.int32, (_N, _G), 1)
    sel = (batch_ref[...] == bi).astype(jnp.float32)
    psum = lax.dot_general(sel, x2, (((0,), (0,)), ((), ())),
                           preferred_element_type=jnp.float32)
    cnt = lax.dot_general(sel, jnp.ones((_N, 1), jnp.float32),
                          (((0,), (0,)), ((), ())),
                          preferred_element_type=jnp.float32)
    pooled = psum / jnp.maximum(cnt, 1.0)
    out_ref[...] = (
        jnp.dot(pooled, wl_ref[...], preferred_element_type=jnp.float32)
        + bl_ref[...])


_post = pl.pallas_call(
    _post_body,
    out_shape=jax.ShapeDtypeStruct((_G, 1), jnp.float32),
)


def kernel(node_features, edge_features, edge_index, batch,
           W_embed, b_embed, W_conv, b_conv, W_lin, b_lin):
    src = edge_index[0].astype(jnp.int32)
    dst = edge_index[1].astype(jnp.int32)
    pad = _EPAD - src.shape[0]
    src_p = jnp.concatenate([src, jnp.zeros((pad,), jnp.int32)])
    src_p = src_p.reshape(_NC, _NS, _CHB, _CH)
    dst_p = jnp.concatenate([dst, jnp.full((pad,), _N, jnp.int32)])
    dst_p = dst_p.reshape(_NC, _NS, _CHB, _CH)

    zeros_deg = jnp.zeros((_NPAD,), jnp.float32)
    deg_parts = _deg_kernel(dst_p, zeros_deg).reshape(_NW, _NR, _CH)

    xw, dinv80 = _dense(node_features, W_embed, b_embed.reshape(1, _D),
                        W_conv, deg_parts)
    dinv = dinv80.reshape(_NPAD, 1)[:_N]
    y = _scale(xw, dinv)

    zeros_agg = jnp.zeros((_NPAD, _D), jnp.float32)
    agg_parts = _agg_kernel(y, src_p, dst_p, zeros_agg)

    out = _post(agg_parts[0], agg_parts[1], xw, dinv,
                b_conv.reshape(1, _D), batch.astype(jnp.int32).reshape(_N, 1),
                W_lin, b_lin.reshape(1, 1))
    return out
